# baseline (speedup 1.0000x reference)
import functools

import jax
import jax.numpy as jnp
from jax import lax
from jax.experimental import pallas as pl
from jax.experimental.pallas import tpu as pltpu
from jax.experimental.pallas import tpu_sc as plsc

_B = 4096
_S = 200
_D = 64
_N = _B * _S
_NW = 32
_PER_W = _N // _NW
_C = 512
_NCH = _PER_W // _C


def _gather_body(idx_hbm, table_hbm, out_hbm, idx_v, rows_v, sem):
    wid = lax.axis_index("s") * 2 + lax.axis_index("c")
    base = wid * _PER_W

    def chunk(i, carry):
        off = base + i * _C
        pltpu.sync_copy(idx_hbm.at[pl.ds(off, _C)], idx_v)
        pltpu.async_copy(table_hbm.at[idx_v], rows_v, sem).wait()
        pltpu.sync_copy(rows_v, out_hbm.at[pl.ds(off, _C)])
        return carry

    lax.fori_loop(0, _NCH, chunk, 0)


_gather = functools.partial(
    pl.kernel,
    out_type=jax.ShapeDtypeStruct((_N, _D), jnp.float32),
    mesh=plsc.VectorSubcoreMesh(core_axis_name="c", subcore_axis_name="s"),
    compiler_params=pltpu.CompilerParams(use_tc_tiling_on_sc=False),
    scratch_types=[
        pltpu.VMEM((_C,), jnp.int32),
        pltpu.VMEM((_C, _D), jnp.float32),
        pltpu.SemaphoreType.DMA,
    ],
)(_gather_body)


def _mask_body(idx_ref, mask_ref):
    mask_ref[...] = (idx_ref[...] != 0).astype(jnp.uint8)


def _mask(input_var):
    return pl.pallas_call(
        _mask_body,
        out_shape=jax.ShapeDtypeStruct((_B, _S), jnp.uint8),
    )(input_var)


def kernel(input_var, W):
    idx = input_var.reshape(-1).astype(jnp.int32)
    emb = _gather(idx, W).reshape(_B, _S, _D)
    mask = _mask(input_var)
    return (emb, emb, mask)


# speedup vs baseline: 1.0425x; 1.0425x over previous
import functools

import jax
import jax.numpy as jnp
from jax import lax
from jax.experimental import pallas as pl
from jax.experimental.pallas import tpu as pltpu
from jax.experimental.pallas import tpu_sc as plsc

_B = 4096
_S = 200
_D = 64
_N = _B * _S
_NW = 32
_PER_W = _N // _NW
_C = 256
_NCH = _PER_W // _C
_NB = 4
_R = _NCH // _NB


def _gather_body(idx_hbm, table_hbm, out_hbm, idx_v,
                 r0, r1, r2, r3, g0, g1, g2, g3, w0, w1, w2, w3):
    rows = (r0, r1, r2, r3)
    gsem = (g0, g1, g2, g3)
    wsem = (w0, w1, w2, w3)
    wid = lax.axis_index("s") * 2 + lax.axis_index("c")
    base = wid * _PER_W

    pltpu.sync_copy(idx_hbm.at[wid], idx_v)
    for b in range(_NB):
        pltpu.async_copy(table_hbm.at[idx_v.at[b]], rows[b], gsem[b])

    def rnd(i, carry):
        for b in range(_NB):
            g = i * _NB + b
            pltpu.make_async_copy(table_hbm.at[idx_v.at[b]], rows[b],
                                  gsem[b]).wait()
            pltpu.async_copy(rows[b], out_hbm.at[pl.ds(base + g * _C, _C)],
                             wsem[b])
            pltpu.make_async_copy(rows[b],
                                  out_hbm.at[pl.ds(base + g * _C, _C)],
                                  wsem[b]).wait()
            pltpu.async_copy(table_hbm.at[idx_v.at[g + _NB]], rows[b],
                             gsem[b])
        return carry

    lax.fori_loop(0, _R - 1, rnd, 0)

    for b in range(_NB):
        g = (_R - 1) * _NB + b
        pltpu.make_async_copy(table_hbm.at[idx_v.at[b]], rows[b],
                              gsem[b]).wait()
        pltpu.async_copy(rows[b], out_hbm.at[pl.ds(base + g * _C, _C)],
                         wsem[b])
    for b in range(_NB):
        g = (_R - 1) * _NB + b
        pltpu.make_async_copy(rows[b],
                              out_hbm.at[pl.ds(base + g * _C, _C)],
                              wsem[b]).wait()


_gather = functools.partial(
    pl.kernel,
    out_type=jax.ShapeDtypeStruct((_N, _D), jnp.float32),
    mesh=plsc.VectorSubcoreMesh(core_axis_name="c", subcore_axis_name="s"),
    compiler_params=pltpu.CompilerParams(use_tc_tiling_on_sc=False),
    scratch_types=(
        [pltpu.VMEM((_NCH, _C), jnp.int32)]
        + [pltpu.VMEM((_C, _D), jnp.float32)] * _NB
        + [pltpu.SemaphoreType.DMA] * (2 * _NB)
    ),
)(_gather_body)


def _mask_body(idx_ref, mask_ref):
    mask_ref[...] = (idx_ref[...] != 0).astype(jnp.uint8)


def _mask(input_var):
    return pl.pallas_call(
        _mask_body,
        out_shape=jax.ShapeDtypeStruct((_B, _S), jnp.uint8),
    )(input_var)


def kernel(input_var, W):
    idx = input_var.reshape(-1).astype(jnp.int32).reshape(_NW, _NCH, _C)
    emb = _gather(idx, W).reshape(_B, _S, _D)
    mask = _mask(input_var)
    return (emb, emb, mask)
